# baseline (speedup 1.0000x reference)
import jax
import jax.numpy as jnp
from jax.experimental import pallas as pl
from jax.experimental.pallas import tpu as pltpu

M = 4
FEAT = 128
FUS = 256
H1 = 4
C1 = 128
D1 = H1 * C1
OUT = 256
BB = 1024


def _fused_kernel(lang, vid, aud, img, mi,
                  Wl, Wv, Wa, Wi, bl, bv, ba, bi,
                  g1W, g1b, g2W, g2b, s1, s2,
                  lng, lnb, hW1, hb1, hW2, hb2,
                  out_ref):
    f32 = jnp.float32
    bf16 = jnp.bfloat16
    bb = lang.shape[0]

    def dot(a, b):
        return jnp.dot(a.astype(bf16), b.astype(bf16),
                       preferred_element_type=f32)

    xt = [lang[...].T, vid[...].T, aud[...].T, img[...].T]
    Ws = [Wl[...], Wv[...], Wa[...], Wi[...]]
    bs = [bl[...], bv[...], ba[...], bi[...]]
    xs = [dot(Ws[m], xt[m]) + bs[m] for m in range(M)]

    mi_v = mi[...]
    pres = [(mi_v != (m + 1)) for m in range(M)]

    def att_weights(hs, a_src, a_dst, heads, S):

        def hred(v):
            return dot(S, v)
        lg = {}
        for d in range(M):
            for s in range(d, M):
                lg[(d, s)] = jax.nn.sigmoid(hred(hs[d] * hs[s]))
        ws = []
        for d in range(M):
            es = []
            for s in range(M):
                sig = lg[(d, s) if d <= s else (s, d)]
                a = (a_src[s] + a_dst[d]) * sig
                a = jnp.where(a >= 0, a, 0.2 * a)
                if s != d:
                    pm = jnp.logical_and(pres[d], pres[s])
                    a = jnp.where(pm, a, -1e30)
                es.append(a)
            ee = [jnp.exp(e) for e in es]
            rden = 1.0 / (ee[0] + ee[1] + ee[2] + ee[3])
            ws.append([e * rden for e in ee])
        return ws

    def bcast(w, heads, ch):
        return jnp.broadcast_to(
            w[:, None, :], (heads, ch, bb)).reshape(heads * ch, bb)

    def gat(hs, a_src, a_dst, heads, S):
        D = hs[0].shape[0]
        ch = D // heads
        ws = att_weights(hs, a_src, a_dst, heads, S)
        outs = []
        for d in range(M):
            acc = None
            for s in range(M):
                term = bcast(ws[d][s], heads, ch) * hs[s]
                acc = term if acc is None else acc + term
            outs.append(acc)
        return outs

    r1 = [dot(g1W[...], x) for x in xs]
    h1 = [r[0:D1] for r in r1]
    t1 = [r[D1:D1 + 2 * H1] for r in r1]
    o1 = gat(h1, [t[0:H1] for t in t1], [t[H1:2 * H1] for t in t1], H1,
             s1[...])
    g1bias = g1b[...]

    def gelu_exact(v):
        return 0.5 * v * (1.0 + jax.lax.erf(v * 0.7071067811865476))

    o1 = [gelu_exact(o + g1bias) for o in o1]

    r2 = [dot(g2W[...], o) for o in o1]
    h2 = [r[0:FUS] for r in r2]
    t2 = [r[FUS:FUS + 2] for r in r2]
    w2 = att_weights(h2, [t[0:1] for t in t2], [t[1:2] for t in t2], 1,
                     s2[...])
    g2bias = g2b[...]

    acc = None
    for s in range(M):
        wsum = ((w2[0][s] + w2[1][s]) + (w2[2][s] + w2[3][s]))
        term = jnp.broadcast_to(wsum, (FUS, bb)) * h2[s]
        acc = term if acc is None else acc + term
    pooled = acc * 0.25 + g2bias
    mu = jnp.mean(pooled, axis=0, keepdims=True)
    cen = pooled - mu
    var = jnp.mean(cen * cen, axis=0, keepdims=True)
    normed = cen * jax.lax.rsqrt(var + 1e-5) * lng[...] + lnb[...]
    hdn = jnp.maximum(dot(hW1[...], normed) + hb1[...], 0.0)
    out_ref[...] = (dot(hW2[...], hdn) + hb2[...]).T


def _pallas_forward(language, video, audio, image, mi_row,
                    W_language, W_video, W_audio, W_image,
                    b_language, b_video, b_audio, b_image,
                    gat1_W, gat1_b, gat2_W, gat2_b, S1, S2,
                    ln_g, ln_b, h_W1, h_b1, h_W2, h_b2,
                    interpret=False):
    B = language.shape[0]
    grid = (B // BB,)

    def blk(shape):
        return pl.BlockSpec(shape, lambda i: (i,) + (0,) * (len(shape) - 1))

    def rep(a):
        return pl.BlockSpec(a.shape, lambda i: (0,) * a.ndim)

    weights = [W_language, W_video, W_audio, W_image,
               b_language, b_video, b_audio, b_image,
               gat1_W, gat1_b, gat2_W, gat2_b, S1, S2,
               ln_g, ln_b, h_W1, h_b1, h_W2, h_b2]

    mi_spec = pl.BlockSpec((1, BB), lambda i: (0, i))

    in_specs = ([blk((BB, FEAT))] * 4 + [mi_spec]
                + [rep(w) for w in weights])

    return pl.pallas_call(
        _fused_kernel,
        grid=grid,
        in_specs=in_specs,
        out_specs=blk((BB, OUT)),
        out_shape=jax.ShapeDtypeStruct((B, OUT), jnp.float32),
        compiler_params=pltpu.CompilerParams(
            dimension_semantics=("parallel",)),
        interpret=interpret,
    )(language, video, audio, image, mi_row, *weights)


def kernel(language, video, audio, image, missing_index,
           W_language, b_language, W_video, b_video,
           W_audio, b_audio, W_image, b_image,
           gat1_W, gat1_al, gat1_ar, gat1_b,
           gat2_W, gat2_al, gat2_ar, gat2_b,
           ln_g, ln_b, h_W1, h_b1, h_W2, h_b2):
    B = language.shape[0]
    col = lambda v: v.reshape(-1, 1)
    g1Wt = gat1_W.T
    g1w3 = g1Wt.reshape(H1, C1, FUS)
    P1 = jnp.concatenate(
        [jnp.einsum("hc,hck->hk", gat1_al.reshape(H1, C1), g1w3),
         jnp.einsum("hc,hck->hk", gat1_ar.reshape(H1, C1), g1w3)], axis=0)
    G1 = jnp.concatenate([g1Wt, P1], axis=0)
    g2Wt = gat2_W.T
    P2 = jnp.stack([gat2_al.reshape(FUS) @ g2Wt,
                    gat2_ar.reshape(FUS) @ g2Wt], axis=0)
    G2 = jnp.concatenate([g2Wt, P2], axis=0)
    S1 = (jnp.arange(D1, dtype=jnp.int32)[None, :] // C1
          == jnp.arange(H1, dtype=jnp.int32)[:, None]).astype(jnp.float32)
    S2 = jnp.ones((1, FUS), jnp.float32)
    return _pallas_forward(
        language, video, audio, image, missing_index.reshape(1, B),
        W_language.T, W_video.T, W_audio.T, W_image.T,
        col(b_language), col(b_video), col(b_audio), col(b_image),
        G1, col(gat1_b), G2, col(gat2_b), S1, S2,
        col(ln_g), col(ln_b), h_W1.T, col(h_b1), h_W2.T, col(h_b2))


# speedup vs baseline: 1.0887x; 1.0887x over previous
import jax
import jax.numpy as jnp
from jax.experimental import pallas as pl
from jax.experimental.pallas import tpu as pltpu

M = 4
FEAT = 128
FUS = 256
H1 = 4
C1 = 128
D1 = H1 * C1
OUT = 256
BB = 1024


def _fused_kernel(lang, vid, aud, img, mi,
                  Wl, Wv, Wa, Wi, bl, bv, ba, bi,
                  g1W, g1b, g2W, g2b,
                  lng, lnb, hW1, hb1, hW2, hb2,
                  out_ref):
    f32 = jnp.float32
    bf16 = jnp.bfloat16
    bb = lang.shape[0]

    def dot(a, b):
        return jnp.dot(a.astype(bf16), b.astype(bf16),
                       preferred_element_type=f32)

    xt = [lang[...].T, vid[...].T, aud[...].T, img[...].T]
    Ws = [Wl[...], Wv[...], Wa[...], Wi[...]]
    bs = [bl[...], bv[...], ba[...], bi[...]]
    xs = [dot(Ws[m], xt[m]) + bs[m] for m in range(M)]

    mi_v = mi[...]
    pres = [(mi_v != (m + 1)) for m in range(M)]

    def att_weights(hs, a_src, a_dst, heads):
        D = hs[0].shape[0]
        ch = D // heads

        def hred(v):
            return v.reshape(heads, ch, bb).sum(axis=1)
        lg = {}
        for d in range(M):
            for s in range(d, M):
                lg[(d, s)] = jax.nn.sigmoid(hred(hs[d] * hs[s]))
        ws = []
        for d in range(M):
            es = []
            for s in range(M):
                sig = lg[(d, s) if d <= s else (s, d)]
                a = (a_src[s] + a_dst[d]) * sig
                a = jnp.where(a >= 0, a, 0.2 * a)
                if s != d:
                    pm = jnp.logical_and(pres[d], pres[s])
                    a = jnp.where(pm, a, -1e30)
                es.append(a)
            ee = [jnp.exp(e) for e in es]
            rden = 1.0 / (ee[0] + ee[1] + ee[2] + ee[3])
            ws.append([e * rden for e in ee])
        return ws

    def bcast(w, heads, ch):
        return jnp.broadcast_to(
            w[:, None, :], (heads, ch, bb)).reshape(heads * ch, bb)

    def gat(hs, a_src, a_dst, heads):
        D = hs[0].shape[0]
        ch = D // heads
        ws = att_weights(hs, a_src, a_dst, heads)
        outs = []
        for d in range(M):
            acc = None
            for s in range(M):
                term = bcast(ws[d][s], heads, ch) * hs[s]
                acc = term if acc is None else acc + term
            outs.append(acc)
        return outs

    r1 = [dot(g1W[...], x) for x in xs]
    h1 = [r[0:D1] for r in r1]
    t1 = [r[D1:D1 + 2 * H1] for r in r1]
    o1 = gat(h1, [t[0:H1] for t in t1], [t[H1:2 * H1] for t in t1], H1)
    g1bias = g1b[...]

    def gelu_exact(v):
        return 0.5 * v * (1.0 + jax.lax.erf(v * 0.7071067811865476))

    o1 = [gelu_exact(o + g1bias) for o in o1]

    r2 = [dot(g2W[...], o) for o in o1]
    h2 = [r[0:FUS] for r in r2]
    t2 = [r[FUS:FUS + 2] for r in r2]
    w2 = att_weights(h2, [t[0:1] for t in t2], [t[1:2] for t in t2], 1)
    g2bias = g2b[...]

    acc = None
    for s in range(M):
        wsum = ((w2[0][s] + w2[1][s]) + (w2[2][s] + w2[3][s]))
        term = jnp.broadcast_to(wsum, (FUS, bb)) * h2[s]
        acc = term if acc is None else acc + term
    pooled = acc * 0.25 + g2bias
    mu = jnp.mean(pooled, axis=0, keepdims=True)
    cen = pooled - mu
    var = jnp.mean(cen * cen, axis=0, keepdims=True)
    normed = cen * jax.lax.rsqrt(var + 1e-5) * lng[...] + lnb[...]
    hdn = jnp.maximum(dot(hW1[...], normed) + hb1[...], 0.0)
    out_ref[...] = (dot(hW2[...], hdn) + hb2[...]).T


def _pallas_forward(language, video, audio, image, mi_row,
                    W_language, W_video, W_audio, W_image,
                    b_language, b_video, b_audio, b_image,
                    gat1_W, gat1_b, gat2_W, gat2_b,
                    ln_g, ln_b, h_W1, h_b1, h_W2, h_b2):
    B = language.shape[0]
    grid = (B // BB,)

    def blk(shape):
        return pl.BlockSpec(shape, lambda i: (i,) + (0,) * (len(shape) - 1))

    def rep(a):
        return pl.BlockSpec(a.shape, lambda i: (0,) * a.ndim)

    weights = [W_language, W_video, W_audio, W_image,
               b_language, b_video, b_audio, b_image,
               gat1_W, gat1_b, gat2_W, gat2_b,
               ln_g, ln_b, h_W1, h_b1, h_W2, h_b2]

    mi_spec = pl.BlockSpec((1, BB), lambda i: (0, i))

    in_specs = ([blk((BB, FEAT))] * 4 + [mi_spec]
                + [rep(w) for w in weights])

    return pl.pallas_call(
        _fused_kernel,
        grid=grid,
        in_specs=in_specs,
        out_specs=blk((BB, OUT)),
        out_shape=jax.ShapeDtypeStruct((B, OUT), jnp.float32),
        compiler_params=pltpu.CompilerParams(
            dimension_semantics=("parallel",)),
    )(language, video, audio, image, mi_row, *weights)


def kernel(language, video, audio, image, missing_index,
           W_language, b_language, W_video, b_video,
           W_audio, b_audio, W_image, b_image,
           gat1_W, gat1_al, gat1_ar, gat1_b,
           gat2_W, gat2_al, gat2_ar, gat2_b,
           ln_g, ln_b, h_W1, h_b1, h_W2, h_b2):
    B = language.shape[0]
    col = lambda v: v.reshape(-1, 1)
    g1Wt = gat1_W.T
    g1w3 = g1Wt.reshape(H1, C1, FUS)
    P1 = jnp.concatenate(
        [jnp.einsum("hc,hck->hk", gat1_al.reshape(H1, C1), g1w3),
         jnp.einsum("hc,hck->hk", gat1_ar.reshape(H1, C1), g1w3)], axis=0)
    G1 = jnp.concatenate([g1Wt, P1], axis=0)
    g2Wt = gat2_W.T
    P2 = jnp.stack([gat2_al.reshape(FUS) @ g2Wt,
                    gat2_ar.reshape(FUS) @ g2Wt], axis=0)
    G2 = jnp.concatenate([g2Wt, P2], axis=0)
    return _pallas_forward(
        language, video, audio, image, missing_index.reshape(1, B),
        W_language.T, W_video.T, W_audio.T, W_image.T,
        col(b_language), col(b_video), col(b_audio), col(b_image),
        G1, col(gat1_b), G2, col(gat2_b),
        col(ln_g), col(ln_b), h_W1.T, col(h_b1), h_W2.T, col(h_b2))
